# baseline (speedup 1.0000x reference)
import functools

import jax
import jax.numpy as jnp
from jax import lax
from jax.experimental import pallas as pl
from jax.experimental.pallas import tpu as pltpu
from jax.experimental.pallas import tpu_sc as plsc

VOCAB = 100000
EMBED = 64
CHORD_START = 1000
CHORD_END = 5000
B, S = 4096, 50

TOKS = B * S
NC, NS, L = 2, 16, 16
NW = NC * NS
TPW = TOKS // NW
CHUNK = 128
NCHUNK = TPW // CHUNK
NBUF = 10
NROUND = NCHUNK // NBUF

COMBO_ZROW = 104
COMBO_ROWS = 112

POST_ROWS = 32


def _tc_combo_body(root_ref, type_ref, w2_ref, w3_ref, b_ref, out_ref):
    rp = jax.lax.dot_general(root_ref[:], w2_ref[:], (((1,), (1,)), ((), ())),
                             preferred_element_type=jnp.float32)
    tp = jax.lax.dot_general(type_ref[:], w3_ref[:], (((1,), (1,)), ((), ())),
                             preferred_element_type=jnp.float32)
    tpb = tp + b_ref[:]
    for r in range(13):
        out_ref[8 * r:8 * r + 8, :] = rp[r:r + 1, :] + tpb
    out_ref[COMBO_ZROW:COMBO_ROWS, :] = jnp.zeros(
        (COMBO_ROWS - COMBO_ZROW, EMBED), jnp.float32)


_tc_combo = pl.pallas_call(
    _tc_combo_body,
    out_shape=jax.ShapeDtypeStruct((COMBO_ROWS, EMBED), jnp.float32),
)


def _sc_gather_body(ids_hbm, table_hbm, out_hbm, ids_v, buf, gsem, ssem):
    wid = lax.axis_index("s") * NC + lax.axis_index("c")
    K = NBUF // 2

    pltpu.sync_copy(ids_hbm.at[wid], ids_v)

    for bslot in range(K):
        pltpu.async_copy(table_hbm.at[ids_v.at[bslot]], buf.at[bslot],
                         gsem.at[bslot])

    def do_round(r, carry):
        for bslot in range(NBUF):
            j = r * NBUF + bslot
            bb = buf.at[bslot]
            pslot = (bslot + K) % NBUF
            pbb = buf.at[pslot]
            pltpu.make_async_copy(table_hbm.at[ids_v.at[j]], bb,
                                  gsem.at[bslot]).wait()

            @pl.when(j >= K)
            def _drain():
                pltpu.make_async_copy(
                    pbb, out_hbm.at[pl.ds((wid * NCHUNK + j - K) * CHUNK,
                                          CHUNK)], ssem.at[pslot]).wait()

            @pl.when(j + K < NCHUNK)
            def _prefetch():
                pltpu.async_copy(table_hbm.at[ids_v.at[j + K]], pbb,
                                 gsem.at[pslot])

            pltpu.async_copy(bb, out_hbm.at[pl.ds((wid * NCHUNK + j) * CHUNK,
                                                  CHUNK)], ssem.at[bslot])
        return carry

    lax.fori_loop(0, NROUND, do_round, 0)

    for bslot in range(NBUF - K, NBUF):
        j = (NROUND - 1) * NBUF + bslot
        pltpu.make_async_copy(
            buf.at[bslot],
            out_hbm.at[pl.ds((wid * NCHUNK + j) * CHUNK, CHUNK)],
            ssem.at[bslot]).wait()


_sc_gather = functools.partial(
    pl.kernel,
    out_type=jax.ShapeDtypeStruct((TOKS, EMBED), jnp.float32),
    mesh=plsc.VectorSubcoreMesh(core_axis_name="c", subcore_axis_name="s"),
    compiler_params=pltpu.CompilerParams(use_tc_tiling_on_sc=False),
    scratch_types=[
        pltpu.VMEM((NCHUNK, CHUNK), jnp.int32),
        pltpu.VMEM((NBUF, CHUNK, EMBED), jnp.float32),
        pltpu.SemaphoreType.DMA((NBUF,)),
        pltpu.SemaphoreType.DMA((NBUF,)),
    ],
)(_sc_gather_body)


def _tc_post_body(rows_ref, ids_ref, roots_ref, types_ref, combo_ref, w1_ref,
                  out_ref):
    g = rows_ref[:]
    tid = ids_ref[:]
    is_chord = (tid >= CHORD_START) & (tid <= CHORD_END)
    cidx = jnp.where(is_chord, roots_ref[:] * 8 + types_ref[:], COMBO_ZROW)
    kidx = jax.lax.broadcasted_iota(jnp.int32, (POST_ROWS, 128, COMBO_ROWS), 2)
    one_hot = (cidx[:, :, None] == kidx).astype(jnp.float32)
    contrib = jax.lax.dot_general(
        one_hot, combo_ref[:], (((2,), (0,)), ((), ())),
        preferred_element_type=jnp.float32)
    proj = jax.lax.dot_general(g, w1_ref[:], (((2,), (0,)), ((), ())),
                               preferred_element_type=jnp.float32)
    m = jax.lax.broadcast_in_dim(is_chord.astype(jnp.float32),
                                 (POST_ROWS, 128, EMBED), (0, 1))
    out_ref[:] = g + m * (proj + contrib - g)


_tc_post = pl.pallas_call(
    _tc_post_body,
    grid=(TOKS // (POST_ROWS * 128),),
    in_specs=[
        pl.BlockSpec((POST_ROWS, 128, EMBED), lambda i: (i, 0, 0)),
        pl.BlockSpec((POST_ROWS, 128), lambda i: (i, 0)),
        pl.BlockSpec((POST_ROWS, 128), lambda i: (i, 0)),
        pl.BlockSpec((POST_ROWS, 128), lambda i: (i, 0)),
        pl.BlockSpec((COMBO_ROWS, EMBED), lambda i: (0, 0)),
        pl.BlockSpec((EMBED, EMBED), lambda i: (0, 0)),
    ],
    out_specs=pl.BlockSpec((POST_ROWS, 128, EMBED), lambda i: (i, 0, 0)),
    out_shape=jax.ShapeDtypeStruct((TOKS // 128, 128, EMBED), jnp.float32),
)


def kernel(token_ids, chord_root_ids, chord_type_ids, token_table, root_table,
           type_table, W, b):
    ids3d = token_ids.astype(jnp.int32).reshape(NW, NCHUNK, CHUNK)
    ids2d = token_ids.astype(jnp.int32).reshape(TOKS // 128, 128)
    roots2d = chord_root_ids.astype(jnp.int32).reshape(TOKS // 128, 128)
    types2d = chord_type_ids.astype(jnp.int32).reshape(TOKS // 128, 128)

    w1 = lax.slice(W, (0, 0), (EMBED, EMBED))
    w2 = lax.slice(W, (0, EMBED), (EMBED, 2 * EMBED))
    w3 = lax.slice(W, (0, 2 * EMBED), (EMBED, 3 * EMBED))
    root_pad = jnp.pad(root_table, ((0, 16 - root_table.shape[0]), (0, 0)))

    combo = _tc_combo(root_pad, type_table, w2, w3, b.reshape(1, EMBED))
    rows = _sc_gather(ids3d, token_table).reshape(TOKS // 128, 128, EMBED)
    out = _tc_post(rows, ids2d, roots2d, types2d, combo, w1.T)
    return out.reshape(B, S, EMBED)
